# Initial kernel scaffold; baseline (speedup 1.0000x reference)
import jax
import jax.numpy as jnp
from jax.experimental import pallas as pl


def kernel(x, edge_index, W1, b1, W2, b2):
    raise NotImplementedError("write your pallas kernel here")


# speedup vs baseline: 24.9565x; 24.9565x over previous
import functools

import jax
import jax.numpy as jnp
from jax import lax
from jax.experimental import pallas as pl
from jax.experimental.pallas import tpu as pltpu
from jax.experimental.pallas import tpu_sc as plsc

N = 10000
E = 320000
D_IN = 128
D_HID = 128
D_CLS = 40
D_CLS_PAD = 64

NC = 2
NS = 16
NW = NC * NS
ET = E // NW
CH = 125
K = ET // CH
RPT = N // NS

_mesh = lambda: plsc.VectorSubcoreMesh(
    core_axis_name="c", subcore_axis_name="s", num_cores=NC, num_subcores=NS)


def _deg_body(dst_hbm, out_hbm, dst_v, deg_v):
    c = lax.axis_index("c")
    s = lax.axis_index("s")
    wid = s * NC + c
    pltpu.sync_copy(dst_hbm.at[pl.ds(wid * ET, ET)], dst_v)

    zeros16 = jnp.zeros((16,), jnp.float32)

    def zero_step(i, _):
        deg_v[pl.ds(i * 16, 16)] = zeros16
        return 0

    lax.fori_loop(0, N // 16, zero_step, 0)

    ones16 = jnp.ones((16,), jnp.float32)

    def acc_step(i, _):
        idx = dst_v[pl.ds(i * 16, 16)]
        plsc.addupdate_scatter(deg_v, [idx], ones16)
        return 0

    lax.fori_loop(0, ET // 16, acc_step, 0)
    pltpu.sync_copy(deg_v, out_hbm.at[wid])


def _deg_partials(dst):
    return pl.kernel(
        _deg_body,
        out_type=jax.ShapeDtypeStruct((NW, N), jnp.float32),
        mesh=_mesh(),
        scratch_types=[
            pltpu.VMEM((ET,), jnp.int32),
            pltpu.VMEM((N,), jnp.float32),
        ],
        compiler_params=pltpu.CompilerParams(needs_layout_passes=False),
    )(dst)


def _agg_body(d, xs_hbm, src_hbm, dst_hbm, zrows_hbm, out_hbm,
              src_v, dst_v, rows_v, acc_sh, sem):
    c = lax.axis_index("c")
    s = lax.axis_index("s")
    wid = s * NC + c
    pltpu.sync_copy(src_hbm.at[wid], src_v)
    pltpu.sync_copy(dst_hbm.at[wid], dst_v)
    pltpu.sync_copy(zrows_hbm, acc_sh.at[pl.ds(s * RPT, RPT)])
    plsc.subcore_barrier()

    def step(k, _):
        pltpu.async_copy(xs_hbm.at[src_v.at[k]], rows_v, sem).wait()
        pltpu.sync_copy(rows_v, acc_sh.at[dst_v.at[k]], add=True)
        return 0

    lax.fori_loop(0, K, step, 0)
    plsc.subcore_barrier()
    pltpu.sync_copy(acc_sh.at[pl.ds(s * RPT, RPT)],
                    out_hbm.at[c, pl.ds(s * RPT, RPT)])


def _edge_aggregate(xs, src3, dst3, d):
    zrows = jnp.zeros((RPT, d), jnp.float32)
    return pl.kernel(
        functools.partial(_agg_body, d),
        out_type=jax.ShapeDtypeStruct((NC, N, d), jnp.float32),
        mesh=_mesh(),
        scratch_types=[
            pltpu.VMEM((K, CH), jnp.int32),
            pltpu.VMEM((K, CH), jnp.int32),
            pltpu.VMEM((CH, d), jnp.float32),
            pltpu.VMEM_SHARED((N, d), jnp.float32),
            pltpu.SemaphoreType.DMA,
        ],
        compiler_params=pltpu.CompilerParams(
            needs_layout_passes=False, use_tc_tiling_on_sc=False),
    )(xs, src3, dst3, zrows)


_BR = 1000


def _scale_mm_body(degt_ref, x_ref, w_ref, xs_ref, dinv_ref):
    deg = jnp.sum(degt_ref[...], axis=1, keepdims=True) + 1.0
    dinv = lax.rsqrt(deg)
    xw = jnp.dot(x_ref[...], w_ref[...], preferred_element_type=jnp.float32)
    xs_ref[...] = dinv * xw
    dinv_ref[...] = dinv


def _scale_mm(degt, x, w1):
    return pl.pallas_call(
        _scale_mm_body,
        grid=(N // _BR,),
        in_specs=[
            pl.BlockSpec((_BR, NW), lambda i: (i, 0)),
            pl.BlockSpec((_BR, D_IN), lambda i: (i, 0)),
            pl.BlockSpec((D_IN, D_HID), lambda i: (0, 0)),
        ],
        out_specs=[
            pl.BlockSpec((_BR, D_HID), lambda i: (i, 0)),
            pl.BlockSpec((_BR, 1), lambda i: (i, 0)),
        ],
        out_shape=[
            jax.ShapeDtypeStruct((N, D_HID), jnp.float32),
            jax.ShapeDtypeStruct((N, 1), jnp.float32),
        ],
    )(degt, x, w1)


def _mid_body(agg_ref, xs_ref, dinv_ref, b1_ref, w2_ref, hs_ref):
    agg = agg_ref[0] + agg_ref[1]
    dinv = dinv_ref[...]
    h = jnp.maximum(dinv * (agg + xs_ref[...]) + b1_ref[...], 0.0)
    hw = jnp.dot(h, w2_ref[...], preferred_element_type=jnp.float32)
    hs_ref[...] = dinv * hw


def _mid(agg, xs, dinv, b1, w2p):
    return pl.pallas_call(
        _mid_body,
        grid=(N // _BR,),
        in_specs=[
            pl.BlockSpec((NC, _BR, D_HID), lambda i: (0, i, 0)),
            pl.BlockSpec((_BR, D_HID), lambda i: (i, 0)),
            pl.BlockSpec((_BR, 1), lambda i: (i, 0)),
            pl.BlockSpec((1, D_HID), lambda i: (0, 0)),
            pl.BlockSpec((D_HID, D_CLS_PAD), lambda i: (0, 0)),
        ],
        out_specs=pl.BlockSpec((_BR, D_CLS_PAD), lambda i: (i, 0)),
        out_shape=jax.ShapeDtypeStruct((N, D_CLS_PAD), jnp.float32),
    )(agg, xs, dinv, b1, w2p)


def _out_body(agg_ref, hs_ref, dinv_ref, b2_ref, o_ref):
    agg = agg_ref[0] + agg_ref[1]
    o = dinv_ref[...] * (agg + hs_ref[...]) + b2_ref[...]
    col = lax.broadcasted_iota(jnp.int32, (_BR, D_CLS_PAD), 1)
    mask = col < D_CLS
    m = jnp.max(jnp.where(mask, o, -jnp.inf), axis=1, keepdims=True)
    e = jnp.where(mask, jnp.exp(o - m), 0.0)
    ssum = jnp.sum(e, axis=1, keepdims=True)
    o_ref[...] = o - m - jnp.log(ssum)


def _final(agg, hs, dinv, b2p):
    return pl.pallas_call(
        _out_body,
        grid=(N // _BR,),
        in_specs=[
            pl.BlockSpec((NC, _BR, D_CLS_PAD), lambda i: (0, i, 0)),
            pl.BlockSpec((_BR, D_CLS_PAD), lambda i: (i, 0)),
            pl.BlockSpec((_BR, 1), lambda i: (i, 0)),
            pl.BlockSpec((1, D_CLS_PAD), lambda i: (0, 0)),
        ],
        out_specs=pl.BlockSpec((_BR, D_CLS_PAD), lambda i: (i, 0)),
        out_shape=jax.ShapeDtypeStruct((N, D_CLS_PAD), jnp.float32),
    )(agg, hs, dinv, b2p)


def kernel(x, edge_index, W1, b1, W2, b2):
    src = edge_index[0].astype(jnp.int32)
    dst = edge_index[1].astype(jnp.int32)
    src3 = src.reshape(NW, K, CH)
    dst3 = dst.reshape(NW, K, CH)

    deg_parts = _deg_partials(dst)
    degt = deg_parts.T

    xs, dinv = _scale_mm(degt, x.astype(jnp.float32), W1)
    agg1 = _edge_aggregate(xs, src3, dst3, D_HID)

    w2p = jnp.pad(W2, ((0, 0), (0, D_CLS_PAD - D_CLS)))
    b1r = b1.reshape(1, D_HID)
    hs = _mid(agg1, xs, dinv, b1r, w2p)

    agg2 = _edge_aggregate(hs, src3, dst3, D_CLS_PAD)
    b2p = jnp.pad(b2, (0, D_CLS_PAD - D_CLS)).reshape(1, D_CLS_PAD)
    out = _final(agg2, hs, dinv, b2p)
    return out[:, :D_CLS]


# speedup vs baseline: 28.7997x; 1.1540x over previous
import functools

import jax
import jax.numpy as jnp
from jax import lax
from jax.experimental import pallas as pl
from jax.experimental.pallas import tpu as pltpu
from jax.experimental.pallas import tpu_sc as plsc

N = 10000
E = 320000
D_IN = 128
D_HID = 128
D_CLS = 40
D_CLS_PAD = 48

NC = 2
NS = 16
NW = NC * NS
ET = E // NW
CH = 100
K = ET // CH
RPT = N // NS

_mesh = lambda: plsc.VectorSubcoreMesh(
    core_axis_name="c", subcore_axis_name="s", num_cores=NC, num_subcores=NS)


def _deg_body(dst_hbm, out_hbm, dst_v, deg_v):
    c = lax.axis_index("c")
    s = lax.axis_index("s")
    wid = s * NC + c
    pltpu.sync_copy(dst_hbm.at[pl.ds(wid * ET, ET)], dst_v)

    zeros16 = jnp.zeros((16,), jnp.float32)

    def zero_step(i, _):
        deg_v[pl.ds(i * 16, 16)] = zeros16
        return 0

    lax.fori_loop(0, N // 16, zero_step, 0)

    ones16 = jnp.ones((16,), jnp.float32)

    def acc_step(i, _):
        idx = dst_v[pl.ds(i * 16, 16)]
        plsc.addupdate_scatter(deg_v, [idx], ones16)
        return 0

    lax.fori_loop(0, ET // 16, acc_step, 0)
    pltpu.sync_copy(deg_v, out_hbm.at[wid])


def _deg_partials(dst):
    return pl.kernel(
        _deg_body,
        out_type=jax.ShapeDtypeStruct((NW, N), jnp.float32),
        mesh=_mesh(),
        scratch_types=[
            pltpu.VMEM((ET,), jnp.int32),
            pltpu.VMEM((N,), jnp.float32),
        ],
        compiler_params=pltpu.CompilerParams(needs_layout_passes=False),
    )(dst)


def _agg_body(d, xs_hbm, src_hbm, dst_hbm, zrows_hbm, out_hbm,
              src_v, dst_v, rows0, rows1, acc_sh, sem0, sem1):
    c = lax.axis_index("c")
    s = lax.axis_index("s")
    wid = s * NC + c
    pltpu.sync_copy(src_hbm.at[wid], src_v)
    pltpu.sync_copy(dst_hbm.at[wid], dst_v)
    pltpu.sync_copy(zrows_hbm, acc_sh.at[pl.ds(s * RPT, RPT)])
    plsc.subcore_barrier()

    pltpu.async_copy(xs_hbm.at[src_v.at[0]], rows0, sem0)

    def step(j, _):
        k0 = 2 * j
        k1 = 2 * j + 1
        pltpu.make_async_copy(xs_hbm.at[src_v.at[k0]], rows0, sem0).wait()
        pltpu.async_copy(xs_hbm.at[src_v.at[k1]], rows1, sem1)
        pltpu.sync_copy(rows0, acc_sh.at[dst_v.at[k0]], add=True)
        pltpu.make_async_copy(xs_hbm.at[src_v.at[k1]], rows1, sem1).wait()

        @pl.when(j < K // 2 - 1)
        def _():
            pltpu.async_copy(xs_hbm.at[src_v.at[k0 + 2]], rows0, sem0)

        pltpu.sync_copy(rows1, acc_sh.at[dst_v.at[k1]], add=True)
        return 0

    lax.fori_loop(0, K // 2, step, 0)
    plsc.subcore_barrier()
    pltpu.sync_copy(acc_sh.at[pl.ds(s * RPT, RPT)],
                    out_hbm.at[c, pl.ds(s * RPT, RPT)])


def _edge_aggregate(xs, src3, dst3, d):
    zrows = jnp.zeros((RPT, d), jnp.float32)
    return pl.kernel(
        functools.partial(_agg_body, d),
        out_type=jax.ShapeDtypeStruct((NC, N, d), jnp.float32),
        mesh=_mesh(),
        scratch_types=[
            pltpu.VMEM((K, CH), jnp.int32),
            pltpu.VMEM((K, CH), jnp.int32),
            pltpu.VMEM((CH, d), jnp.float32),
            pltpu.VMEM((CH, d), jnp.float32),
            pltpu.VMEM_SHARED((N, d), jnp.float32),
            pltpu.SemaphoreType.DMA,
            pltpu.SemaphoreType.DMA,
        ],
        compiler_params=pltpu.CompilerParams(
            needs_layout_passes=False, use_tc_tiling_on_sc=False),
    )(xs, src3, dst3, zrows)


_BR = 1000


def _scale_mm_body(degt_ref, x_ref, w_ref, xs_ref, dinv_ref):
    deg = jnp.sum(degt_ref[...], axis=1, keepdims=True) + 1.0
    dinv = lax.rsqrt(deg)
    xw = jnp.dot(x_ref[...], w_ref[...], preferred_element_type=jnp.float32)
    xs_ref[...] = dinv * xw
    dinv_ref[...] = dinv


def _scale_mm(degt, x, w1):
    return pl.pallas_call(
        _scale_mm_body,
        grid=(N // _BR,),
        in_specs=[
            pl.BlockSpec((_BR, NW), lambda i: (i, 0)),
            pl.BlockSpec((_BR, D_IN), lambda i: (i, 0)),
            pl.BlockSpec((D_IN, D_HID), lambda i: (0, 0)),
        ],
        out_specs=[
            pl.BlockSpec((_BR, D_HID), lambda i: (i, 0)),
            pl.BlockSpec((_BR, 1), lambda i: (i, 0)),
        ],
        out_shape=[
            jax.ShapeDtypeStruct((N, D_HID), jnp.float32),
            jax.ShapeDtypeStruct((N, 1), jnp.float32),
        ],
    )(degt, x, w1)


def _mid_body(agg_ref, xs_ref, dinv_ref, b1_ref, w2_ref, hs_ref):
    agg = agg_ref[0] + agg_ref[1]
    dinv = dinv_ref[...]
    h = jnp.maximum(dinv * (agg + xs_ref[...]) + b1_ref[...], 0.0)
    hw = jnp.dot(h, w2_ref[...], preferred_element_type=jnp.float32)
    hs_ref[...] = dinv * hw


def _mid(agg, xs, dinv, b1, w2p):
    return pl.pallas_call(
        _mid_body,
        grid=(N // _BR,),
        in_specs=[
            pl.BlockSpec((NC, _BR, D_HID), lambda i: (0, i, 0)),
            pl.BlockSpec((_BR, D_HID), lambda i: (i, 0)),
            pl.BlockSpec((_BR, 1), lambda i: (i, 0)),
            pl.BlockSpec((1, D_HID), lambda i: (0, 0)),
            pl.BlockSpec((D_HID, D_CLS_PAD), lambda i: (0, 0)),
        ],
        out_specs=pl.BlockSpec((_BR, D_CLS_PAD), lambda i: (i, 0)),
        out_shape=jax.ShapeDtypeStruct((N, D_CLS_PAD), jnp.float32),
    )(agg, xs, dinv, b1, w2p)


def _out_body(agg_ref, hs_ref, dinv_ref, b2_ref, o_ref):
    agg = agg_ref[0] + agg_ref[1]
    o = dinv_ref[...] * (agg + hs_ref[...]) + b2_ref[...]
    col = lax.broadcasted_iota(jnp.int32, (_BR, D_CLS_PAD), 1)
    mask = col < D_CLS
    m = jnp.max(jnp.where(mask, o, -jnp.inf), axis=1, keepdims=True)
    e = jnp.where(mask, jnp.exp(o - m), 0.0)
    ssum = jnp.sum(e, axis=1, keepdims=True)
    o_ref[...] = o - m - jnp.log(ssum)


def _final(agg, hs, dinv, b2p):
    return pl.pallas_call(
        _out_body,
        grid=(N // _BR,),
        in_specs=[
            pl.BlockSpec((NC, _BR, D_CLS_PAD), lambda i: (0, i, 0)),
            pl.BlockSpec((_BR, D_CLS_PAD), lambda i: (i, 0)),
            pl.BlockSpec((_BR, 1), lambda i: (i, 0)),
            pl.BlockSpec((1, D_CLS_PAD), lambda i: (0, 0)),
        ],
        out_specs=pl.BlockSpec((_BR, D_CLS_PAD), lambda i: (i, 0)),
        out_shape=jax.ShapeDtypeStruct((N, D_CLS_PAD), jnp.float32),
    )(agg, hs, dinv, b2p)


def kernel(x, edge_index, W1, b1, W2, b2):
    src = edge_index[0].astype(jnp.int32)
    dst = edge_index[1].astype(jnp.int32)
    src3 = src.reshape(NW, K, CH)
    dst3 = dst.reshape(NW, K, CH)

    deg_parts = _deg_partials(dst)
    degt = deg_parts.T

    xs, dinv = _scale_mm(degt, x.astype(jnp.float32), W1)
    agg1 = _edge_aggregate(xs, src3, dst3, D_HID)

    w2p = jnp.pad(W2, ((0, 0), (0, D_CLS_PAD - D_CLS)))
    b1r = b1.reshape(1, D_HID)
    hs = _mid(agg1, xs, dinv, b1r, w2p)

    agg2 = _edge_aggregate(hs, src3, dst3, D_CLS_PAD)
    b2p = jnp.pad(b2, (0, D_CLS_PAD - D_CLS)).reshape(1, D_CLS_PAD)
    out = _final(agg2, hs, dinv, b2p)
    return out[:, :D_CLS]


# speedup vs baseline: 29.5552x; 1.0262x over previous
import functools

import jax
import jax.numpy as jnp
from jax import lax
from jax.experimental import pallas as pl
from jax.experimental.pallas import tpu as pltpu
from jax.experimental.pallas import tpu_sc as plsc

N = 10000
E = 320000
D_IN = 128
D_HID = 128
D_CLS = 40
D_CLS_PAD = 48

NC = 2
NS = 16
NW = NC * NS
ET = E // NW
CH = 100
K = ET // CH
RPT = N // NS

_mesh = lambda: plsc.VectorSubcoreMesh(
    core_axis_name="c", subcore_axis_name="s", num_cores=NC, num_subcores=NS)


def _deg_body(dst_hbm, out_hbm, dst_v, deg_v):
    c = lax.axis_index("c")
    s = lax.axis_index("s")
    wid = s * NC + c
    pltpu.sync_copy(dst_hbm.at[pl.ds(wid * ET, ET)], dst_v)

    zeros16 = jnp.zeros((16,), jnp.float32)

    def zero_step(i, _):
        deg_v[pl.ds(i * 16, 16)] = zeros16
        return 0

    lax.fori_loop(0, N // 16, zero_step, 0)

    ones16 = jnp.ones((16,), jnp.float32)

    def acc_step(i, _):
        idx = dst_v[pl.ds(i * 16, 16)]
        plsc.addupdate_scatter(deg_v, [idx], ones16)
        return 0

    lax.fori_loop(0, ET // 16, acc_step, 0)
    pltpu.sync_copy(deg_v, out_hbm.at[wid])


def _deg_partials(dst):
    return pl.kernel(
        _deg_body,
        out_type=jax.ShapeDtypeStruct((NW, N), jnp.float32),
        mesh=_mesh(),
        scratch_types=[
            pltpu.VMEM((ET,), jnp.int32),
            pltpu.VMEM((N,), jnp.float32),
        ],
        compiler_params=pltpu.CompilerParams(needs_layout_passes=False),
    )(dst)


def _agg_body(d, xs_hbm, src_hbm, dst_hbm, zrows_hbm, out_hbm,
              src_v, dst_v, rows0, rows1, acc_sh, sem0, sem1):
    c = lax.axis_index("c")
    s = lax.axis_index("s")
    wid = s * NC + c
    pltpu.sync_copy(src_hbm.at[wid], src_v)
    pltpu.sync_copy(dst_hbm.at[wid], dst_v)
    pltpu.sync_copy(zrows_hbm, acc_sh.at[pl.ds(s * RPT, RPT)])
    plsc.subcore_barrier()

    pltpu.async_copy(xs_hbm.at[src_v.at[0]], rows0, sem0)

    def step(j, _):
        k0 = 2 * j
        k1 = 2 * j + 1
        pltpu.make_async_copy(xs_hbm.at[src_v.at[k0]], rows0, sem0).wait()
        pltpu.async_copy(xs_hbm.at[src_v.at[k1]], rows1, sem1)
        pltpu.sync_copy(rows0, acc_sh.at[dst_v.at[k0]], add=True)
        pltpu.make_async_copy(xs_hbm.at[src_v.at[k1]], rows1, sem1).wait()

        @pl.when(j < K // 2 - 1)
        def _():
            pltpu.async_copy(xs_hbm.at[src_v.at[k0 + 2]], rows0, sem0)

        pltpu.sync_copy(rows1, acc_sh.at[dst_v.at[k1]], add=True)
        return 0

    lax.fori_loop(0, K // 2, step, 0)
    plsc.subcore_barrier()
    pltpu.sync_copy(acc_sh.at[pl.ds(s * RPT, RPT)],
                    out_hbm.at[c, pl.ds(s * RPT, RPT)])


def _edge_aggregate(xs, src3, dst3, d):
    zrows = jnp.zeros((RPT, d), jnp.float32)
    return pl.kernel(
        functools.partial(_agg_body, d),
        out_type=jax.ShapeDtypeStruct((NC, N, d), jnp.float32),
        mesh=_mesh(),
        scratch_types=[
            pltpu.VMEM((K, CH), jnp.int32),
            pltpu.VMEM((K, CH), jnp.int32),
            pltpu.VMEM((CH, d), jnp.float32),
            pltpu.VMEM((CH, d), jnp.float32),
            pltpu.VMEM_SHARED((N, d), jnp.float32),
            pltpu.SemaphoreType.DMA,
            pltpu.SemaphoreType.DMA,
        ],
        compiler_params=pltpu.CompilerParams(
            needs_layout_passes=False, use_tc_tiling_on_sc=False),
    )(xs, src3, dst3, zrows)


_BR = 1000


def _scale_mm_body(degp_ref, x_ref, w_ref, xs_ref, dinv_ref):
    ones_col = jnp.ones((NW, 1), jnp.float32)
    deg = lax.dot_general(degp_ref[...], ones_col, (((0,), (0,)), ((), ())),
                          preferred_element_type=jnp.float32) + 1.0
    dinv = lax.rsqrt(deg)
    xw = jnp.dot(x_ref[...], w_ref[...], preferred_element_type=jnp.float32)
    xs_ref[...] = dinv * xw
    dinv_ref[...] = dinv


def _scale_mm(degp, x, w1):
    return pl.pallas_call(
        _scale_mm_body,
        out_shape=[
            jax.ShapeDtypeStruct((N, D_HID), jnp.float32),
            jax.ShapeDtypeStruct((N, 1), jnp.float32),
        ],
    )(degp, x, w1)


def _mid_body(agg_ref, xs_ref, dinv_ref, b1_ref, w2_ref, hs_ref):
    agg = agg_ref[0] + agg_ref[1]
    dinv = dinv_ref[...]
    h = jnp.maximum(dinv * (agg + xs_ref[...]) + b1_ref[...], 0.0)
    hw = jnp.dot(h, w2_ref[...], preferred_element_type=jnp.float32)
    hs_ref[...] = dinv * hw


def _mid(agg, xs, dinv, b1, w2p):
    return pl.pallas_call(
        _mid_body,
        grid=(N // _BR,),
        in_specs=[
            pl.BlockSpec((NC, _BR, D_HID), lambda i: (0, i, 0)),
            pl.BlockSpec((_BR, D_HID), lambda i: (i, 0)),
            pl.BlockSpec((_BR, 1), lambda i: (i, 0)),
            pl.BlockSpec((1, D_HID), lambda i: (0, 0)),
            pl.BlockSpec((D_HID, D_CLS_PAD), lambda i: (0, 0)),
        ],
        out_specs=pl.BlockSpec((_BR, D_CLS_PAD), lambda i: (i, 0)),
        out_shape=jax.ShapeDtypeStruct((N, D_CLS_PAD), jnp.float32),
    )(agg, xs, dinv, b1, w2p)


def _out_body(agg_ref, hs_ref, dinv_ref, b2_ref, o_ref):
    agg = agg_ref[0] + agg_ref[1]
    o = dinv_ref[...] * (agg + hs_ref[...]) + b2_ref[...]
    col = lax.broadcasted_iota(jnp.int32, (_BR, D_CLS_PAD), 1)
    mask = col < D_CLS
    m = jnp.max(jnp.where(mask, o, -jnp.inf), axis=1, keepdims=True)
    e = jnp.where(mask, jnp.exp(o - m), 0.0)
    ssum = jnp.sum(e, axis=1, keepdims=True)
    o_ref[...] = (o - m - jnp.log(ssum))[:, :D_CLS]


def _final(agg, hs, dinv, b2p):
    return pl.pallas_call(
        _out_body,
        grid=(N // _BR,),
        in_specs=[
            pl.BlockSpec((NC, _BR, D_CLS_PAD), lambda i: (0, i, 0)),
            pl.BlockSpec((_BR, D_CLS_PAD), lambda i: (i, 0)),
            pl.BlockSpec((_BR, 1), lambda i: (i, 0)),
            pl.BlockSpec((1, D_CLS_PAD), lambda i: (0, 0)),
        ],
        out_specs=pl.BlockSpec((_BR, D_CLS), lambda i: (i, 0)),
        out_shape=jax.ShapeDtypeStruct((N, D_CLS), jnp.float32),
    )(agg, hs, dinv, b2p)


def kernel(x, edge_index, W1, b1, W2, b2):
    src = edge_index[0].astype(jnp.int32)
    dst = edge_index[1].astype(jnp.int32)
    src3 = src.reshape(NW, K, CH)
    dst3 = dst.reshape(NW, K, CH)

    deg_parts = _deg_partials(dst)

    xs, dinv = _scale_mm(deg_parts, x.astype(jnp.float32), W1)
    agg1 = _edge_aggregate(xs, src3, dst3, D_HID)

    w2p = jnp.pad(W2, ((0, 0), (0, D_CLS_PAD - D_CLS)))
    b1r = b1.reshape(1, D_HID)
    hs = _mid(agg1, xs, dinv, b1r, w2p)

    agg2 = _edge_aggregate(hs, src3, dst3, D_CLS_PAD)
    b2p = jnp.pad(b2, (0, D_CLS_PAD - D_CLS)).reshape(1, D_CLS_PAD)
    return _final(agg2, hs, dinv, b2p)
